# baseline (speedup 1.0000x reference)
import jax
import jax.numpy as jnp
from jax import lax
from jax.experimental import pallas as pl
from jax.experimental.pallas import tpu as pltpu
from jax.experimental.pallas import tpu_sc as plsc

_EMB = 64
_L = 200
_SCALE = 8.0

_NC = 2
_NS = 16
_NW = _NC * _NS
_BB = 128


_NG = 8
_NT = 2


def _emb_body(tokens_hbm, table_hbm, out_hbm,
              idx_v, gbufs, tbufs, gsems, osems):
    wid = lax.axis_index("s") * _NC + lax.axis_index("c")

    pltpu.sync_copy(tokens_hbm.at[wid], idx_v)

    def dbl(i, c):
        for j in range(_BB // 16):
            sl = pl.ds(j * 16, 16)
            idx_v[i, sl] = idx_v[i, sl] * 2
        return c

    lax.fori_loop(0, _L, dbl, 0)

    def start_gather(l, p):
        pltpu.async_copy(table_hbm.at[idx_v.at[l]], gbufs[p], gsems[p])

    def wait_gather(p):
        pltpu.make_async_copy(table_hbm.at[pl.ds(0, _BB)], gbufs[p],
                              gsems[p]).wait()

    def start_out(l, p):
        for e8 in range(_EMB // 8):
            pltpu.async_copy(tbufs[p].at[pl.ds(e8 * 8, 8)],
                             out_hbm.at[l, e8, wid], osems[p])

    def wait_out(p):
        for e8 in range(_EMB // 8):
            pltpu.make_async_copy(tbufs[p].at[pl.ds(e8 * 8, 8)],
                                  out_hbm.at[0, e8, wid], osems[p]).wait()

    rows = [lax.iota(jnp.int32, 16) + (bj * 16) for bj in range(_BB // 16)]

    def transpose_scale(gp, tp):
        src, dst = gbufs[gp], tbufs[tp]

        def col(e4, c):
            for de in range(4):
                e = e4 * 4 + de
                ev = jnp.full((16,), e, jnp.int32)
                for bj in range(_BB // 16):
                    v = plsc.load_gather(src, [rows[bj], ev])
                    dst[e, pl.ds(bj * 16, 16)] = v * _SCALE
            return c

        lax.fori_loop(0, _EMB // 4, col, 0)

    for p in range(_NG):
        start_gather(p, p)

    def outer(s, carry):
        l0 = s * _NG
        for p in range(_NG):
            l = l0 + p
            wait_gather(p)
            tp = p % _NT

            @pl.when(l >= _NT)
            def _():
                wait_out(tp)
            transpose_scale(p, tp)
            start_out(l, tp)

            @pl.when(l + _NG < _L)
            def _():
                start_gather(l + _NG, p)
        return carry

    lax.fori_loop(0, _L // _NG, outer, 0)
    wait_out(0)
    wait_out(1)


def kernel(tokens, table):
    b, l = tokens.shape
    tokens_arr = tokens.T.reshape(l, _NW, _BB).transpose(1, 0, 2)
    table_pad = jnp.pad(table, ((0, 0), (0, 64))).reshape(-1, _EMB)
    mesh = plsc.VectorSubcoreMesh(core_axis_name="c", subcore_axis_name="s")
    out5 = pl.kernel(
        _emb_body,
        out_type=jax.ShapeDtypeStruct((l, _EMB // 8, _NW, 8, _BB),
                                      jnp.float32),
        mesh=mesh,
        scratch_types=[
            pltpu.VMEM((_L, _BB), jnp.int32),
            [pltpu.VMEM((_BB, _EMB), jnp.float32) for _ in range(_NG)],
            [pltpu.VMEM((_EMB, _BB), jnp.float32) for _ in range(_NT)],
            [pltpu.SemaphoreType.DMA for _ in range(_NG)],
            [pltpu.SemaphoreType.DMA for _ in range(_NT)],
        ],
        compiler_params=pltpu.CompilerParams(use_tc_tiling_on_sc=False,
                                             needs_layout_passes=False),
    )(tokens_arr, table_pad)
    return out5.transpose(2, 4, 0, 1, 3).reshape(b, l, _EMB)


# speedup vs baseline: 1.2178x; 1.2178x over previous
import jax
import jax.numpy as jnp
from jax import lax
from jax.experimental import pallas as pl
from jax.experimental.pallas import tpu as pltpu
from jax.experimental.pallas import tpu_sc as plsc

_EMB = 64
_L = 200
_SCALE = 8.0

_NC = 2
_NS = 16
_NW = _NC * _NS
_BB = 128


_NG = 8
_NT = 2


def _emb_body(tokens_hbm, table_hbm, out_hbm,
              idx_v, gbufs, tbufs, gsems, osems):
    wid = lax.axis_index("s") * _NC + lax.axis_index("c")

    pltpu.sync_copy(tokens_hbm.at[wid], idx_v)

    def dbl(i, c):
        for j in range(_BB // 16):
            sl = pl.ds(j * 16, 16)
            idx_v[i, sl] = idx_v[i, sl] * 2
        return c

    lax.fori_loop(0, _L, dbl, 0)

    def start_gather(l, p):
        pltpu.async_copy(table_hbm.at[idx_v.at[l]], gbufs[p], gsems[p])

    def wait_gather(p):
        pltpu.make_async_copy(table_hbm.at[pl.ds(0, _BB)], gbufs[p],
                              gsems[p]).wait()

    def start_out(l, p):
        for e8 in range(_EMB // 8):
            pltpu.async_copy(tbufs[p].at[pl.ds(e8 * 8, 8)],
                             out_hbm.at[l, e8, wid], osems[p])

    def wait_out(p):
        for e8 in range(_EMB // 8):
            pltpu.make_async_copy(tbufs[p].at[pl.ds(e8 * 8, 8)],
                                  out_hbm.at[0, e8, wid], osems[p]).wait()

    rows = [lax.iota(jnp.int32, 16) + (bj * 16) for bj in range(_BB // 16)]

    def transpose_scale(gp, tp):
        src, dst = gbufs[gp], tbufs[tp]

        def col(e4, c):
            for de in range(4):
                e = e4 * 4 + de
                ev = jnp.full((16,), e, jnp.int32)
                vs = [plsc.load_gather(src, [rows[bj], ev])
                      for bj in range(_BB // 16)]
                for bj in range(_BB // 16):
                    dst[e, pl.ds(bj * 16, 16)] = vs[bj] * _SCALE
            return c

        lax.fori_loop(0, _EMB // 4, col, 0)

    for p in range(_NG):
        start_gather(p, p)

    def outer(s, carry):
        l0 = s * _NG
        for p in range(_NG):
            l = l0 + p
            wait_gather(p)
            tp = p % _NT

            @pl.when(l >= _NT)
            def _():
                wait_out(tp)
            transpose_scale(p, tp)
            start_out(l, tp)

            @pl.when(l + _NG < _L)
            def _():
                start_gather(l + _NG, p)
        return carry

    lax.fori_loop(0, _L // _NG, outer, 0)
    wait_out(0)
    wait_out(1)


def kernel(tokens, table):
    b, l = tokens.shape
    tokens_arr = tokens.T.reshape(l, _NW, _BB).transpose(1, 0, 2)
    table_pad = jnp.pad(table, ((0, 0), (0, 64))).reshape(-1, _EMB)
    mesh = plsc.VectorSubcoreMesh(core_axis_name="c", subcore_axis_name="s")
    out5 = pl.kernel(
        _emb_body,
        out_type=jax.ShapeDtypeStruct((l, _EMB // 8, _NW, 8, _BB),
                                      jnp.float32),
        mesh=mesh,
        scratch_types=[
            pltpu.VMEM((_L, _BB), jnp.int32),
            [pltpu.VMEM((_BB, _EMB), jnp.float32) for _ in range(_NG)],
            [pltpu.VMEM((_EMB, _BB), jnp.float32) for _ in range(_NT)],
            [pltpu.SemaphoreType.DMA for _ in range(_NG)],
            [pltpu.SemaphoreType.DMA for _ in range(_NT)],
        ],
        compiler_params=pltpu.CompilerParams(use_tc_tiling_on_sc=False,
                                             needs_layout_passes=False),
    )(tokens_arr, table_pad)
    return out5.transpose(2, 4, 0, 1, 3).reshape(b, l, _EMB)


# speedup vs baseline: 1.2185x; 1.0006x over previous
import jax
import jax.numpy as jnp
from jax import lax
from jax.experimental import pallas as pl
from jax.experimental.pallas import tpu as pltpu
from jax.experimental.pallas import tpu_sc as plsc

_EMB = 64
_L = 200
_SCALE = 8.0

_NC = 2
_NS = 16
_NW = _NC * _NS
_BB = 128


_NG = 8
_NT = 2


def _emb_body(tokens_hbm, table_hbm, out_hbm,
              idx_v, gbufs, tbufs, gsems, osems):
    wid = lax.axis_index("s") * _NC + lax.axis_index("c")

    pltpu.sync_copy(tokens_hbm.at[wid], idx_v)

    def dbl(i, c):
        for j in range(_BB // 16):
            sl = pl.ds(j * 16, 16)
            idx_v[i, sl] = idx_v[i, sl] * 2
        return c

    lax.fori_loop(0, _L, dbl, 0)

    def start_gather(l, p):
        pltpu.async_copy(table_hbm.at[idx_v.at[l]], gbufs[p], gsems[p])

    def wait_gather(p):
        pltpu.make_async_copy(table_hbm.at[pl.ds(0, _BB)], gbufs[p],
                              gsems[p]).wait()

    def start_out(l, p):
        pltpu.async_copy(tbufs[p], out_hbm.at[l, :, wid], osems[p])

    def wait_out(p):
        pltpu.make_async_copy(tbufs[p], out_hbm.at[0, :, wid],
                              osems[p]).wait()

    rows = [lax.iota(jnp.int32, 16) + (bj * 16) for bj in range(_BB // 16)]

    def transpose_scale(gp, tp):
        src, dst = gbufs[gp], tbufs[tp]

        def col(e8, c):
            for ee in range(8):
                ev = jnp.full((16,), e8 * 8 + ee, jnp.int32)
                vs = [plsc.load_gather(src, [rows[bj], ev])
                      for bj in range(_BB // 16)]
                for bj in range(_BB // 16):
                    dst[e8, ee, pl.ds(bj * 16, 16)] = vs[bj] * _SCALE
            return c

        lax.fori_loop(0, _EMB // 8, col, 0)

    for p in range(_NG):
        start_gather(p, p)

    def outer(s, carry):
        l0 = s * _NG
        for p in range(_NG):
            l = l0 + p
            wait_gather(p)
            tp = p % _NT

            @pl.when(l >= _NT)
            def _():
                wait_out(tp)
            transpose_scale(p, tp)
            start_out(l, tp)

            @pl.when(l + _NG < _L)
            def _():
                start_gather(l + _NG, p)
        return carry

    lax.fori_loop(0, _L // _NG, outer, 0)
    wait_out(0)
    wait_out(1)


def kernel(tokens, table):
    b, l = tokens.shape
    tokens_arr = tokens.T.reshape(l, _NW, _BB).transpose(1, 0, 2)
    table_pad = jnp.pad(table, ((0, 0), (0, 64))).reshape(-1, _EMB)
    mesh = plsc.VectorSubcoreMesh(core_axis_name="c", subcore_axis_name="s")
    out5 = pl.kernel(
        _emb_body,
        out_type=jax.ShapeDtypeStruct((l, _EMB // 8, _NW, 8, _BB),
                                      jnp.float32),
        mesh=mesh,
        scratch_types=[
            pltpu.VMEM((_L, _BB), jnp.int32),
            [pltpu.VMEM((_BB, _EMB), jnp.float32) for _ in range(_NG)],
            [pltpu.VMEM((_EMB // 8, 8, _BB), jnp.float32) for _ in range(_NT)],
            [pltpu.SemaphoreType.DMA for _ in range(_NG)],
            [pltpu.SemaphoreType.DMA for _ in range(_NT)],
        ],
        compiler_params=pltpu.CompilerParams(use_tc_tiling_on_sc=False,
                                             needs_layout_passes=False),
    )(tokens_arr, table_pad)
    return out5.transpose(2, 4, 0, 1, 3).reshape(b, l, _EMB)
